# baseline (speedup 1.0000x reference)
import jax
import jax.numpy as jnp
from jax import lax
from jax.experimental import pallas as pl
from jax.experimental.pallas import tpu as pltpu

B = 16
L = 4096
D = 256
S = 32


def _resample_tc(table, lens_s, lens_v, out, pidx_v, pidx_s, pairs, sem_i,
                 sem):
    l = jnp.broadcast_to(lens_v[...], (B, S))
    l_f = l.astype(jnp.float32)
    k = lax.broadcasted_iota(jnp.int32, (B, S), 1)
    g = jnp.minimum(l & (-l), S)
    g_f = g.astype(jnp.float32)
    step_f = l_f / g_f
    j_f = k.astype(jnp.float32) * step_f
    scale = g_f * (1.0 / S)
    pos = (j_f + 0.5) * scale - 0.5
    pos = jnp.minimum(jnp.maximum(pos, 0.0), l_f - 1.0)
    lo = pos.astype(jnp.int32)
    hi = jnp.minimum(lo + 1, l - 1)
    w = pos - lo.astype(jnp.float32)
    p = jnp.maximum(jnp.minimum(lo, l - 2), 0)
    c1 = (jnp.where(lo == p + 1, 1.0 - w, 0.0)
          + jnp.where(hi == p + 1, w, 0.0))

    brow = lax.broadcasted_iota(jnp.int32, (B, S), 0)
    pidx_v[...] = (p + brow * L) * D

    idx_cp = pltpu.make_async_copy(pidx_v, pidx_s, sem_i)
    idx_cp.start()
    idx_cp.wait()

    copies = []
    for b in range(B):
        for kk in range(S):
            pg = pl.multiple_of(pidx_s[b, kk], D)
            cp = pltpu.make_async_copy(
                table.at[pl.ds(pg, 2 * D)],
                pairs.at[b, kk],
                sem,
            )
            cp.start()
            copies.append(cp)

    for cp in copies:
        cp.wait()

    for kk in range(S):
        p0 = pairs[:, kk, pl.ds(0, D)]
        p1 = pairs[:, kk, pl.ds(D, D)]
        ck = c1[:, kk:kk + 1]
        out[:, pl.ds(kk * D, D)] = p0 + ck * (p1 - p0)
    out[:, pl.ds(S * D, 1)] = lens_v[...].astype(jnp.float32)


def kernel(padded_input, lengths):
    table = padded_input.reshape(B * L * D)
    lens = lengths.astype(jnp.int32).reshape(B, 1)
    return pl.pallas_call(
        _resample_tc,
        in_specs=[
            pl.BlockSpec(memory_space=pltpu.MemorySpace.HBM),
            pl.BlockSpec(memory_space=pltpu.MemorySpace.SMEM),
            pl.BlockSpec(memory_space=pltpu.MemorySpace.VMEM),
        ],
        out_specs=pl.BlockSpec(memory_space=pltpu.MemorySpace.VMEM),
        out_shape=jax.ShapeDtypeStruct((B, S * D + 1), jnp.float32),
        scratch_shapes=[
            pltpu.VMEM((B, S), jnp.int32),
            pltpu.SMEM((B, S), jnp.int32),
            pltpu.VMEM((B, S, 2 * D), jnp.float32),
            pltpu.SemaphoreType.DMA,
            pltpu.SemaphoreType.DMA,
        ],
    )(table, lens, lens)


# speedup vs baseline: 1.0001x; 1.0001x over previous
import jax
import jax.numpy as jnp
from jax import lax
from jax.experimental import pallas as pl
from jax.experimental.pallas import tpu as pltpu

B = 16
L = 4096
D = 256
S = 32
NSEM = 16


def _resample_tc(table, lens_s, lens_v, out, pidx_v, pidx_s, pairs, sem_i,
                 sem):
    l = jnp.broadcast_to(lens_v[...], (B, S))
    l_f = l.astype(jnp.float32)
    k = lax.broadcasted_iota(jnp.int32, (B, S), 1)
    g = jnp.minimum(l & (-l), S)
    g_f = g.astype(jnp.float32)
    step_f = l_f / g_f
    j_f = k.astype(jnp.float32) * step_f
    scale = g_f * (1.0 / S)
    pos = (j_f + 0.5) * scale - 0.5
    pos = jnp.minimum(jnp.maximum(pos, 0.0), l_f - 1.0)
    lo = pos.astype(jnp.int32)
    hi = jnp.minimum(lo + 1, l - 1)
    w = pos - lo.astype(jnp.float32)
    p = jnp.maximum(jnp.minimum(lo, l - 2), 0)
    c1 = (jnp.where(lo == p + 1, 1.0 - w, 0.0)
          + jnp.where(hi == p + 1, w, 0.0))

    brow = lax.broadcasted_iota(jnp.int32, (B, S), 0)
    pidx_v[...] = (p + brow * L) * D

    idx_cp = pltpu.make_async_copy(pidx_v, pidx_s, sem_i)
    idx_cp.start()
    idx_cp.wait()

    copies = []
    for b in range(B):
        for kk in range(S):
            pg = pl.multiple_of(pidx_s[b, kk], D)
            cp = pltpu.make_async_copy(
                table.at[pl.ds(pg, 2 * D)],
                pairs.at[b, kk],
                sem.at[(b * S + kk) % NSEM],
            )
            cp.start()
            copies.append(cp)

    for cp in copies:
        cp.wait()

    for kk in range(S):
        p0 = pairs[:, kk, pl.ds(0, D)]
        p1 = pairs[:, kk, pl.ds(D, D)]
        ck = c1[:, kk:kk + 1]
        out[:, pl.ds(kk * D, D)] = p0 + ck * (p1 - p0)
    out[:, pl.ds(S * D, 1)] = lens_v[...].astype(jnp.float32)


def kernel(padded_input, lengths):
    table = padded_input.reshape(B * L * D)
    lens = lengths.astype(jnp.int32).reshape(B, 1)
    return pl.pallas_call(
        _resample_tc,
        in_specs=[
            pl.BlockSpec(memory_space=pltpu.MemorySpace.HBM),
            pl.BlockSpec(memory_space=pltpu.MemorySpace.SMEM),
            pl.BlockSpec(memory_space=pltpu.MemorySpace.VMEM),
        ],
        out_specs=pl.BlockSpec(memory_space=pltpu.MemorySpace.VMEM),
        out_shape=jax.ShapeDtypeStruct((B, S * D + 1), jnp.float32),
        scratch_shapes=[
            pltpu.VMEM((B, S), jnp.int32),
            pltpu.SMEM((B, S), jnp.int32),
            pltpu.VMEM((B, S, 2 * D), jnp.float32),
            pltpu.SemaphoreType.DMA,
            pltpu.SemaphoreType.DMA((NSEM,)),
        ],
    )(table, lens, lens)


# speedup vs baseline: 4.2975x; 4.2970x over previous
import jax
import jax.numpy as jnp
from jax import lax
from jax.experimental import pallas as pl
from jax.experimental.pallas import tpu as pltpu

B = 16
L = 4096
D = 256
S = 32
BS = B * S
NROW = 16


def _resample_tc(table, lens_row, lens_col, out, pidx_v, pidx_s, win, sem_i,
                 sem):
    t = lax.broadcasted_iota(jnp.int32, (BS, 1), 0)
    b_id = t & (B - 1)
    k_id = t >> 4
    onehot = b_id == lax.broadcasted_iota(jnp.int32, (BS, B), 1)
    l_row = jnp.broadcast_to(lens_row[...], (BS, B))
    l = jnp.sum(jnp.where(onehot, l_row, 0), axis=1, keepdims=True)

    l_f = l.astype(jnp.float32)
    g = jnp.minimum(l & (-l), S)
    g_f = g.astype(jnp.float32)
    step_f = l_f / g_f
    j_f = k_id.astype(jnp.float32) * step_f
    scale = g_f * (1.0 / S)
    pos = (j_f + 0.5) * scale - 0.5
    pos = jnp.minimum(jnp.maximum(pos, 0.0), l_f - 1.0)
    lo = pos.astype(jnp.int32)
    hi = jnp.minimum(lo + 1, l - 1)
    w = pos - lo.astype(jnp.float32)

    row0 = b_id * L
    lo_g = row0 + lo
    hi_g = row0 + hi
    p = jnp.maximum(jnp.minimum(lo, l - 2), 0) + row0
    base = jnp.minimum((p >> 3) << 3, B * L - NROW)
    u = lo_g - base
    v = hi_g - base
    pidx_v[...] = base

    r_iota = lax.broadcasted_iota(jnp.int32, (BS, NROW), 1)
    C = (jnp.where(u == r_iota, 1.0 - w, 0.0)
         + jnp.where(v == r_iota, w, 0.0))

    idx_cp = pltpu.make_async_copy(pidx_v, pidx_s, sem_i)
    idx_cp.start()
    idx_cp.wait()

    copies = []
    for tt in range(BS):
        bs = pl.multiple_of(pidx_s[tt, 0], 8)
        cp = pltpu.make_async_copy(
            table.at[pl.ds(bs, NROW), :],
            win.at[tt],
            sem,
        )
        cp.start()
        copies.append(cp)
    for cp in copies:
        cp.wait()

    acc = C[:, 0:1] * win[:, 0, :]
    for r in range(1, NROW):
        acc = acc + C[:, r:r + 1] * win[:, r, :]

    for k in range(S):
        out[:, pl.ds(k * D, D)] = acc[k * B:(k + 1) * B, :]
    out[:, pl.ds(S * D, 1)] = lens_col[...]


def kernel(padded_input, lengths):
    table = padded_input.reshape(B * L, D)
    lens_row = lengths.astype(jnp.int32).reshape(1, B)
    lens_col = lengths.astype(jnp.float32).reshape(B, 1)
    return pl.pallas_call(
        _resample_tc,
        in_specs=[
            pl.BlockSpec(memory_space=pltpu.MemorySpace.HBM),
            pl.BlockSpec(memory_space=pltpu.MemorySpace.VMEM),
            pl.BlockSpec(memory_space=pltpu.MemorySpace.VMEM),
        ],
        out_specs=pl.BlockSpec(memory_space=pltpu.MemorySpace.VMEM),
        out_shape=jax.ShapeDtypeStruct((B, S * D + 1), jnp.float32),
        scratch_shapes=[
            pltpu.VMEM((BS, 1), jnp.int32),
            pltpu.SMEM((BS, 1), jnp.int32),
            pltpu.VMEM((BS, NROW, D), jnp.float32),
            pltpu.SemaphoreType.DMA,
            pltpu.SemaphoreType.DMA,
        ],
    )(table, lens_row, lens_col)


# speedup vs baseline: 4.6715x; 1.0870x over previous
import jax
import jax.numpy as jnp
from jax import lax
from jax.experimental import pallas as pl
from jax.experimental.pallas import tpu as pltpu

B = 16
L = 4096
D = 256
S = 32
BS = B * S
NROW = 16
NSEM = 16


def _resample_tc(table, lens_row, out, pidx_v, pidx_s, win, sem_i, sem):
    t = lax.broadcasted_iota(jnp.int32, (BS, 1), 0)
    b_id = t & (B - 1)
    k_id = t >> 4
    onehot = b_id == lax.broadcasted_iota(jnp.int32, (BS, B), 1)
    l_row = jnp.broadcast_to(lens_row[...], (BS, B))
    l = jnp.sum(jnp.where(onehot, l_row, 0), axis=1, keepdims=True)

    l_f = l.astype(jnp.float32)
    g = jnp.minimum(l & (-l), S)
    g_f = g.astype(jnp.float32)
    step_f = l_f / g_f
    j_f = k_id.astype(jnp.float32) * step_f
    scale = g_f * (1.0 / S)
    pos = (j_f + 0.5) * scale - 0.5
    pos = jnp.minimum(jnp.maximum(pos, 0.0), l_f - 1.0)
    lo = pos.astype(jnp.int32)
    hi = jnp.minimum(lo + 1, l - 1)
    w = pos - lo.astype(jnp.float32)

    row0 = b_id * L
    lo_g = row0 + lo
    hi_g = row0 + hi
    p = jnp.maximum(jnp.minimum(lo, l - 2), 0) + row0
    base = jnp.minimum((p >> 3) << 3, B * L - NROW)
    u = lo_g - base
    v = hi_g - base
    pidx_v[...] = base

    r_iota = lax.broadcasted_iota(jnp.int32, (BS, NROW), 1)
    C = (jnp.where(u == r_iota, 1.0 - w, 0.0)
         + jnp.where(v == r_iota, w, 0.0))

    idx_cp = pltpu.make_async_copy(pidx_v, pidx_s, sem_i)
    idx_cp.start()
    idx_cp.wait()

    copies = []
    for tt in range(BS):
        bs = pl.multiple_of(pidx_s[tt, 0], 8)
        cp = pltpu.make_async_copy(
            table.at[pl.ds(bs, NROW), :],
            win.at[tt],
            sem.at[tt % NSEM],
        )
        cp.start()
        copies.append(cp)
    for cp in copies:
        cp.wait()

    acc = C[:, 0:1] * win[:, 0, :]
    for r in range(1, NROW):
        acc = acc + C[:, r:r + 1] * win[:, r, :]

    for k in range(S):
        out[:, pl.ds(k * D, D)] = acc[k * B:(k + 1) * B, :]
    out[:, pl.ds(S * D, 1)] = l_f[0:B, :]


def kernel(padded_input, lengths):
    table = padded_input.reshape(B * L, D)
    lens_row = lengths.astype(jnp.int32).reshape(1, B)
    return pl.pallas_call(
        _resample_tc,
        in_specs=[
            pl.BlockSpec(memory_space=pltpu.MemorySpace.HBM),
            pl.BlockSpec(memory_space=pltpu.MemorySpace.VMEM),
        ],
        out_specs=pl.BlockSpec(memory_space=pltpu.MemorySpace.VMEM),
        out_shape=jax.ShapeDtypeStruct((B, S * D + 1), jnp.float32),
        scratch_shapes=[
            pltpu.VMEM((BS, 1), jnp.int32),
            pltpu.SMEM((BS, 1), jnp.int32),
            pltpu.VMEM((BS, NROW, D), jnp.float32),
            pltpu.SemaphoreType.DMA,
            pltpu.SemaphoreType.DMA((NSEM,)),
        ],
    )(table, lens_row)
